# baseline (speedup 1.0000x reference)
import functools

import jax
import jax.numpy as jnp
from jax.experimental import pallas as pl
from jax.experimental.pallas import tpu as pltpu

_BLK_N1 = 512


def _fp_kernel(xyz1_ref, feat1_ref, xyz2t_ref, feat2_ref, W1_ref, b1_ref,
               W2_ref, b2_ref, out_ref, G_scr, *, n2, c2):
    i = pl.program_id(1)

    @pl.when(i == 0)
    def _compute_g():
        G_scr[...] = jnp.dot(feat2_ref[0], W1_ref[:c2, :],
                             preferred_element_type=jnp.float32)

    x1 = xyz1_ref[0]
    x2t = xyz2t_ref[0]
    blk = x1.shape[0]

    x1sq = jnp.sum(x1 * x1, axis=1, keepdims=True)
    x2sq = jnp.sum(x2t * x2t, axis=0, keepdims=True)
    cross = jnp.dot(x1, x2t, preferred_element_type=jnp.float32,
                    precision=jax.lax.Precision.HIGHEST)
    d = jnp.maximum(x1sq + x2sq - 2.0 * cross, 0.0)

    lane = jax.lax.broadcasted_iota(jnp.int32, (blk, n2), 1)
    key = (d.view(jnp.int32) & jnp.int32(~0xFF)) | lane
    imax = jnp.int32(0x7FFFFFFF)
    m1 = jnp.min(key, axis=1, keepdims=True)
    k2 = jnp.where(key == m1, imax, key)
    m2 = jnp.min(k2, axis=1, keepdims=True)
    k3 = jnp.where(k2 == m2, imax, k2)
    m3 = jnp.min(k3, axis=1, keepdims=True)

    nn_mask = key <= m3
    d_sel = (key & jnp.int32(~0xFF)).view(jnp.float32)
    w = jnp.where(nn_mask, 1.0 / jnp.maximum(d_sel, 1e-10), 0.0)
    denom = jnp.sum(w, axis=1, keepdims=True)
    S = w * (1.0 / denom)

    h = jnp.dot(S, G_scr[...], preferred_element_type=jnp.float32)
    h = h + jnp.dot(feat1_ref[0], W1_ref[c2:, :],
                    preferred_element_type=jnp.float32)
    h = jnp.maximum(h + b1_ref[...], 0.0)
    out = jnp.dot(h, W2_ref[...], preferred_element_type=jnp.float32)
    out_ref[0] = jnp.maximum(out + b2_ref[...], 0.0)


@jax.jit
def kernel(xyz1, feat1, xyz2, feat2, W1, b1, W2, b2):
    B, N1, _ = xyz1.shape
    _, N2, C2 = feat2.shape
    C1 = feat1.shape[-1]
    MLP = W2.shape[-1]
    blk = _BLK_N1
    nb = N1 // blk

    xyz2t = jnp.swapaxes(xyz2, 1, 2)
    b1r = b1.reshape(1, MLP)
    b2r = b2.reshape(1, MLP)

    grid = (B, nb)
    out = pl.pallas_call(
        functools.partial(_fp_kernel, n2=N2, c2=C2),
        grid=grid,
        in_specs=[
            pl.BlockSpec((1, blk, 3), lambda b, i: (b, i, 0)),
            pl.BlockSpec((1, blk, C1), lambda b, i: (b, i, 0)),
            pl.BlockSpec((1, 3, N2), lambda b, i: (b, 0, 0)),
            pl.BlockSpec((1, N2, C2), lambda b, i: (b, 0, 0)),
            pl.BlockSpec((C1 + C2, MLP), lambda b, i: (0, 0)),
            pl.BlockSpec((1, MLP), lambda b, i: (0, 0)),
            pl.BlockSpec((MLP, MLP), lambda b, i: (0, 0)),
            pl.BlockSpec((1, MLP), lambda b, i: (0, 0)),
        ],
        out_specs=pl.BlockSpec((1, blk, MLP), lambda b, i: (b, i, 0)),
        out_shape=jax.ShapeDtypeStruct((B, N1, MLP), jnp.float32),
        scratch_shapes=[pltpu.VMEM((N2, MLP), jnp.float32)],
    )(xyz1, feat1, xyz2t, feat2, W1, b1r, W2, b2r)
    return out


# speedup vs baseline: 1.0997x; 1.0997x over previous
import functools

import jax
import jax.numpy as jnp
from jax.experimental import pallas as pl
from jax.experimental.pallas import tpu as pltpu

_BLK_N1 = 512


def _fp_kernel(xyz1_ref, feat1_ref, xyz2t_ref, feat2_ref, W1_ref, b1_ref,
               W2_ref, b2_ref, out_ref, G_scr, *, n2, c2):
    i = pl.program_id(1)

    @pl.when(i == 0)
    def _compute_g():
        G_scr[...] = jnp.dot(feat2_ref[0], W1_ref[:c2, :],
                             preferred_element_type=jnp.float32)

    x1 = xyz1_ref[0]
    x2t = xyz2t_ref[0]
    blk = x1.shape[0]

    d = jnp.zeros((blk, n2), dtype=jnp.float32)
    for k in range(3):
        diff = x1[:, k:k + 1] - x2t[k:k + 1, :]
        d = d + diff * diff

    lane = jax.lax.broadcasted_iota(jnp.int32, (blk, n2), 1)
    key = (d.view(jnp.int32) & jnp.int32(~0xFF)) | lane
    imax = jnp.int32(0x7FFFFFFF)
    m1 = jnp.min(key, axis=1, keepdims=True)
    k2 = jnp.where(key == m1, imax, key)
    m2 = jnp.min(k2, axis=1, keepdims=True)
    k3 = jnp.where(k2 == m2, imax, k2)
    m3 = jnp.min(k3, axis=1, keepdims=True)

    nn_mask = key <= m3
    d_sel = (key & jnp.int32(~0xFF)).view(jnp.float32)
    w = jnp.where(nn_mask, 1.0 / jnp.maximum(d_sel, 1e-10), 0.0)
    denom = jnp.sum(w, axis=1, keepdims=True)
    S = w * (1.0 / denom)

    h = jnp.dot(S, G_scr[...], preferred_element_type=jnp.float32)
    h = h + jnp.dot(feat1_ref[0], W1_ref[c2:, :],
                    preferred_element_type=jnp.float32)
    h = jnp.maximum(h + b1_ref[...], 0.0)
    out = jnp.dot(h, W2_ref[...], preferred_element_type=jnp.float32)
    out_ref[0] = jnp.maximum(out + b2_ref[...], 0.0)


@jax.jit
def kernel(xyz1, feat1, xyz2, feat2, W1, b1, W2, b2):
    B, N1, _ = xyz1.shape
    _, N2, C2 = feat2.shape
    C1 = feat1.shape[-1]
    MLP = W2.shape[-1]
    blk = _BLK_N1
    nb = N1 // blk

    xyz2t = jnp.swapaxes(xyz2, 1, 2)
    b1r = b1.reshape(1, MLP)
    b2r = b2.reshape(1, MLP)

    grid = (B, nb)
    out = pl.pallas_call(
        functools.partial(_fp_kernel, n2=N2, c2=C2),
        grid=grid,
        in_specs=[
            pl.BlockSpec((1, blk, 3), lambda b, i: (b, i, 0)),
            pl.BlockSpec((1, blk, C1), lambda b, i: (b, i, 0)),
            pl.BlockSpec((1, 3, N2), lambda b, i: (b, 0, 0)),
            pl.BlockSpec((1, N2, C2), lambda b, i: (b, 0, 0)),
            pl.BlockSpec((C1 + C2, MLP), lambda b, i: (0, 0)),
            pl.BlockSpec((1, MLP), lambda b, i: (0, 0)),
            pl.BlockSpec((MLP, MLP), lambda b, i: (0, 0)),
            pl.BlockSpec((1, MLP), lambda b, i: (0, 0)),
        ],
        out_specs=pl.BlockSpec((1, blk, MLP), lambda b, i: (b, i, 0)),
        out_shape=jax.ShapeDtypeStruct((B, N1, MLP), jnp.float32),
        scratch_shapes=[pltpu.VMEM((N2, MLP), jnp.float32)],
    )(xyz1, feat1, xyz2t, feat2, W1, b1r, W2, b2r)
    return out


# speedup vs baseline: 1.1183x; 1.0169x over previous
import functools

import jax
import jax.numpy as jnp
from jax.experimental import pallas as pl
from jax.experimental.pallas import tpu as pltpu

_BLK_N1 = 512


def _fp_kernel(xyz1_ref, feat1_ref, xyz2t_ref, feat2_ref, W1_ref, b1_ref,
               W2_ref, b2_ref, out_ref, G_scr, *, n2, c2):
    i = pl.program_id(1)

    @pl.when(i == 0)
    def _compute_g():
        G_scr[...] = jnp.dot(feat2_ref[0], W1_ref[:c2, :],
                             preferred_element_type=jnp.float32)

    x1 = xyz1_ref[0]
    x2t = xyz2t_ref[0]
    blk = x1.shape[0]

    d = jnp.zeros((blk, n2), dtype=jnp.float32)
    for k in range(3):
        diff = x1[:, k:k + 1] - x2t[k:k + 1, :]
        d = d + diff * diff

    inf = jnp.float32(jnp.inf)
    m1 = jnp.min(d, axis=1, keepdims=True)
    k2 = jnp.where(d == m1, inf, d)
    m2 = jnp.min(k2, axis=1, keepdims=True)
    k3 = jnp.where(k2 == m2, inf, k2)
    m3 = jnp.min(k3, axis=1, keepdims=True)

    nn_mask = d <= m3
    w = jnp.where(nn_mask, 1.0 / jnp.maximum(d, 1e-10), 0.0)
    denom = jnp.sum(w, axis=1, keepdims=True)
    S = w * (1.0 / denom)

    h = jnp.dot(S, G_scr[...], preferred_element_type=jnp.float32)
    h = h + jnp.dot(feat1_ref[0], W1_ref[c2:, :],
                    preferred_element_type=jnp.float32)
    h = jnp.maximum(h + b1_ref[...], 0.0)
    out = jnp.dot(h, W2_ref[...], preferred_element_type=jnp.float32)
    out_ref[0] = jnp.maximum(out + b2_ref[...], 0.0)


@jax.jit
def kernel(xyz1, feat1, xyz2, feat2, W1, b1, W2, b2):
    B, N1, _ = xyz1.shape
    _, N2, C2 = feat2.shape
    C1 = feat1.shape[-1]
    MLP = W2.shape[-1]
    blk = _BLK_N1
    nb = N1 // blk

    xyz2t = jnp.swapaxes(xyz2, 1, 2)
    b1r = b1.reshape(1, MLP)
    b2r = b2.reshape(1, MLP)

    grid = (B, nb)
    out = pl.pallas_call(
        functools.partial(_fp_kernel, n2=N2, c2=C2),
        grid=grid,
        in_specs=[
            pl.BlockSpec((1, blk, 3), lambda b, i: (b, i, 0)),
            pl.BlockSpec((1, blk, C1), lambda b, i: (b, i, 0)),
            pl.BlockSpec((1, 3, N2), lambda b, i: (b, 0, 0)),
            pl.BlockSpec((1, N2, C2), lambda b, i: (b, 0, 0)),
            pl.BlockSpec((C1 + C2, MLP), lambda b, i: (0, 0)),
            pl.BlockSpec((1, MLP), lambda b, i: (0, 0)),
            pl.BlockSpec((MLP, MLP), lambda b, i: (0, 0)),
            pl.BlockSpec((1, MLP), lambda b, i: (0, 0)),
        ],
        out_specs=pl.BlockSpec((1, blk, MLP), lambda b, i: (b, i, 0)),
        out_shape=jax.ShapeDtypeStruct((B, N1, MLP), jnp.float32),
        scratch_shapes=[pltpu.VMEM((N2, MLP), jnp.float32)],
    )(xyz1, feat1, xyz2t, feat2, W1, b1r, W2, b2r)
    return out


# speedup vs baseline: 1.7169x; 1.5353x over previous
import functools

import jax
import jax.numpy as jnp
from jax.experimental import pallas as pl
from jax.experimental.pallas import tpu as pltpu

_BLK_N1 = 1024


def _fp_kernel(xyz1_ref, feat1_ref, xyz2t_ref, feat2_ref, W1_ref, b1_ref,
               W2_ref, b2_ref, out_ref, G_scr, *, n2, c2):
    i = pl.program_id(1)

    @pl.when(i == 0)
    def _compute_g():
        G_scr[...] = jnp.dot(feat2_ref[0], W1_ref[:c2, :],
                             preferred_element_type=jnp.float32)

    x1 = xyz1_ref[0]
    x2t = xyz2t_ref[0]
    blk = x1.shape[0]

    d = jnp.zeros((blk, n2), dtype=jnp.float32)
    for k in range(3):
        diff = x1[:, k:k + 1] - x2t[k:k + 1, :]
        d = d + diff * diff

    inf = jnp.float32(jnp.inf)
    m1 = jnp.min(d, axis=1, keepdims=True)
    k2 = jnp.where(d == m1, inf, d)
    m2 = jnp.min(k2, axis=1, keepdims=True)
    k3 = jnp.where(k2 == m2, inf, k2)
    m3 = jnp.min(k3, axis=1, keepdims=True)

    nn_mask = d <= m3
    w = jnp.where(nn_mask, 1.0 / jnp.maximum(d, 1e-10), 0.0)
    denom = jnp.sum(w, axis=1, keepdims=True)
    S = w * (1.0 / denom)

    h = jnp.dot(S, G_scr[...], preferred_element_type=jnp.float32)
    h = h + jnp.dot(feat1_ref[0], W1_ref[c2:, :],
                    preferred_element_type=jnp.float32)
    h = jnp.maximum(h + b1_ref[...], 0.0)
    out = jnp.dot(h, W2_ref[...], preferred_element_type=jnp.float32)
    out_ref[0] = jnp.maximum(out + b2_ref[...], 0.0)


@jax.jit
def kernel(xyz1, feat1, xyz2, feat2, W1, b1, W2, b2):
    B, N1, _ = xyz1.shape
    _, N2, C2 = feat2.shape
    C1 = feat1.shape[-1]
    MLP = W2.shape[-1]
    blk = _BLK_N1
    nb = N1 // blk

    xyz2t = jnp.swapaxes(xyz2, 1, 2)
    b1r = b1.reshape(1, MLP)
    b2r = b2.reshape(1, MLP)

    grid = (B, nb)
    out = pl.pallas_call(
        functools.partial(_fp_kernel, n2=N2, c2=C2),
        grid=grid,
        in_specs=[
            pl.BlockSpec((1, blk, 3), lambda b, i: (b, i, 0)),
            pl.BlockSpec((1, blk, C1), lambda b, i: (b, i, 0)),
            pl.BlockSpec((1, 3, N2), lambda b, i: (b, 0, 0)),
            pl.BlockSpec((1, N2, C2), lambda b, i: (b, 0, 0)),
            pl.BlockSpec((C1 + C2, MLP), lambda b, i: (0, 0)),
            pl.BlockSpec((1, MLP), lambda b, i: (0, 0)),
            pl.BlockSpec((MLP, MLP), lambda b, i: (0, 0)),
            pl.BlockSpec((1, MLP), lambda b, i: (0, 0)),
        ],
        out_specs=pl.BlockSpec((1, blk, MLP), lambda b, i: (b, i, 0)),
        out_shape=jax.ShapeDtypeStruct((B, N1, MLP), jnp.float32),
        scratch_shapes=[pltpu.VMEM((N2, MLP), jnp.float32)],
    )(xyz1, feat1, xyz2t, feat2, W1, b1r, W2, b2r)
    return out
